# baseline (speedup 1.0000x reference)
import functools

import jax
import jax.numpy as jnp
from jax import lax
from jax.experimental import pallas as pl
from jax.experimental.pallas import tpu as pltpu
from jax.experimental.pallas import tpu_sc as plsc

D = 32
K = 8192
BT = 512


def _argmin_body(z_ref, c_ref, cn_ref, idx_ref):
    z = z_ref[...]
    c = c_ref[...]
    cn = cn_ref[...]
    zn = jnp.sum(z * z, axis=1, keepdims=True)
    p = lax.dot_general(-2.0 * z, c, (((1,), (1,)), ((), ())),
                        preferred_element_type=jnp.float32)
    scores = (zn + cn) + p
    idx_ref[0, 0, :] = jnp.argmin(scores, axis=-1).astype(jnp.int32)


def _tc_argmin(zflat, codebook):
    t = zflat.shape[0]
    nb = t // BT
    cn = jnp.sum(codebook ** 2, axis=1).reshape(1, K)
    out = pl.pallas_call(
        _argmin_body,
        grid=(nb,),
        in_specs=[
            pl.BlockSpec((BT, D), lambda i: (i, 0)),
            pl.BlockSpec((K, D), lambda i: (0, 0)),
            pl.BlockSpec((1, K), lambda i: (0, 0)),
        ],
        out_specs=pl.BlockSpec((1, 1, BT), lambda i: (i, 0, 0)),
        out_shape=jax.ShapeDtypeStruct((nb, 1, BT), jnp.int32),
    )(zflat, codebook, cn)
    return out.reshape(t)


DPAD = 128


@functools.lru_cache(maxsize=None)
def _make_sc_gather(t):
    info = plsc.get_sparse_core_info()
    nc, ns = info.num_cores, info.num_subcores
    nw = nc * ns
    bpw = t // nw

    @functools.partial(
        pl.kernel,
        mesh=plsc.VectorSubcoreMesh(core_axis_name="c", subcore_axis_name="s"),
        out_type=jax.ShapeDtypeStruct((t, D), jnp.float32),
        scratch_types=[
            pltpu.VMEM((2, bpw // 2), jnp.int32),
            pltpu.VMEM((2, bpw // 2, D), jnp.float32),
            pltpu.SemaphoreType.DMA,
            pltpu.SemaphoreType.DMA,
        ],
        compiler_params=pltpu.CompilerParams(use_tc_tiling_on_sc=False),
    )
    def gather(cb_hbm, idx_hbm, out_hbm, idx_v, rows_v, sem0, sem1):
        wid = lax.axis_index("s") * nc + lax.axis_index("c")
        half = bpw // 2
        base = wid * bpw
        sems = (sem0, sem1)
        for j in range(2):
            pltpu.sync_copy(idx_hbm.at[pl.ds(base + j * half, half)],
                            idx_v.at[j])
        copies = [
            pltpu.async_copy(cb_hbm.at[idx_v.at[j]], rows_v.at[j], sems[j])
            for j in range(2)
        ]
        for j in range(2):
            copies[j].wait()
            pltpu.sync_copy(rows_v.at[j],
                            out_hbm.at[pl.ds(base + j * half, half)])

    return gather


def kernel(ze, codebook):
    b, s, d = ze.shape
    t = b * s
    zflat = ze.reshape(t, d)
    idx = _tc_argmin(zflat, codebook)
    zq = _make_sc_gather(t)(codebook, idx)
    return zq.reshape(b, s, d)


# speedup vs baseline: 1.0510x; 1.0510x over previous
import functools

import jax
import jax.numpy as jnp
from jax import lax
from jax.experimental import pallas as pl
from jax.experimental.pallas import tpu as pltpu
from jax.experimental.pallas import tpu_sc as plsc

D = 32
K = 8192
BT = 256


NU = 4


def _argmin_body(z_ref, c_ref, cn_ref, idx_ref, *bufs):
    p_bufs, zn_bufs = bufs[:NU], bufs[NU:]
    c = c_ref[...]
    cn = cn_ref[...]

    def mm(j):
        z = z_ref[pl.ds(j * BT, BT), :]
        zn_bufs[j][...] = jnp.sum(z * z, axis=1, keepdims=True)
        p_bufs[j][...] = lax.dot_general(
            -2.0 * z, c, (((1,), (1,)), ((), ())),
            preferred_element_type=jnp.float32)

    def am(j):
        scores = (zn_bufs[j][...] + cn) + p_bufs[j][...]
        idx_ref[0, j, :] = jnp.argmin(scores, axis=-1).astype(jnp.int32)

    mm(0)
    for j in range(1, NU):
        mm(j)
        am(j - 1)
    am(NU - 1)


def _tc_argmin(zflat, codebook):
    t = zflat.shape[0]
    nb = t // (BT * NU)
    cn = jnp.sum(codebook ** 2, axis=1).reshape(1, K)
    out = pl.pallas_call(
        _argmin_body,
        grid=(nb,),
        in_specs=[
            pl.BlockSpec((BT * NU, D), lambda i: (i, 0)),
            pl.BlockSpec((K, D), lambda i: (0, 0)),
            pl.BlockSpec((1, K), lambda i: (0, 0)),
        ],
        out_specs=pl.BlockSpec((1, NU, BT), lambda i: (i, 0, 0)),
        out_shape=jax.ShapeDtypeStruct((nb, NU, BT), jnp.int32),
        scratch_shapes=(
            [pltpu.VMEM((BT, K), jnp.float32) for _ in range(NU)]
            + [pltpu.VMEM((BT, 1), jnp.float32) for _ in range(NU)]
        ),
    )(zflat, codebook, cn)
    return out.reshape(t)


DPAD = 128


@functools.lru_cache(maxsize=None)
def _make_sc_gather(t):
    info = plsc.get_sparse_core_info()
    nc, ns = info.num_cores, info.num_subcores
    nw = nc * ns
    bpw = t // nw

    @functools.partial(
        pl.kernel,
        mesh=plsc.VectorSubcoreMesh(core_axis_name="c", subcore_axis_name="s"),
        out_type=jax.ShapeDtypeStruct((t, D), jnp.float32),
        scratch_types=[
            pltpu.VMEM((2, bpw // 2), jnp.int32),
            pltpu.VMEM((2, bpw // 2, D), jnp.float32),
            pltpu.SemaphoreType.DMA,
            pltpu.SemaphoreType.DMA,
        ],
        compiler_params=pltpu.CompilerParams(use_tc_tiling_on_sc=False),
    )
    def gather(cb_hbm, idx_hbm, out_hbm, idx_v, rows_v, sem0, sem1):
        wid = lax.axis_index("s") * nc + lax.axis_index("c")
        half = bpw // 2
        base = wid * bpw
        sems = (sem0, sem1)
        for j in range(2):
            pltpu.sync_copy(idx_hbm.at[pl.ds(base + j * half, half)],
                            idx_v.at[j])
        copies = [
            pltpu.async_copy(cb_hbm.at[idx_v.at[j]], rows_v.at[j], sems[j])
            for j in range(2)
        ]
        for j in range(2):
            copies[j].wait()
            pltpu.sync_copy(rows_v.at[j],
                            out_hbm.at[pl.ds(base + j * half, half)])

    return gather


def kernel(ze, codebook):
    b, s, d = ze.shape
    t = b * s
    zflat = ze.reshape(t, d)
    idx = _tc_argmin(zflat, codebook)
    zq = _make_sc_gather(t)(codebook, idx)
    return zq.reshape(b, s, d)


# speedup vs baseline: 1.0632x; 1.0116x over previous
import functools

import jax
import jax.numpy as jnp
from jax import lax
from jax.experimental import pallas as pl
from jax.experimental.pallas import tpu as pltpu
from jax.experimental.pallas import tpu_sc as plsc

D = 32
K = 8192
BT = 256


NU = 8


def _argmin_body(z_ref, c_ref, cn_ref, idx_ref, *bufs):
    p_bufs, zn_bufs = bufs[:NU], bufs[NU:]
    c = c_ref[...]
    cn = cn_ref[...]

    def mm(j):
        z = z_ref[pl.ds(j * BT, BT), :]
        zn_bufs[j][...] = jnp.sum(z * z, axis=1, keepdims=True)
        p_bufs[j][...] = lax.dot_general(
            -2.0 * z, c, (((1,), (1,)), ((), ())),
            preferred_element_type=jnp.float32)

    def am(j):
        scores = (zn_bufs[j][...] + cn) + p_bufs[j][...]
        idx_ref[0, j, :] = jnp.argmin(scores, axis=-1).astype(jnp.int32)

    mm(0)
    for j in range(1, NU):
        mm(j)
        am(j - 1)
    am(NU - 1)


def _tc_argmin(zflat, codebook):
    t = zflat.shape[0]
    nb = t // (BT * NU)
    cn = jnp.sum(codebook ** 2, axis=1).reshape(1, K)
    out = pl.pallas_call(
        _argmin_body,
        grid=(nb,),
        in_specs=[
            pl.BlockSpec((BT * NU, D), lambda i: (i, 0)),
            pl.BlockSpec((K, D), lambda i: (0, 0)),
            pl.BlockSpec((1, K), lambda i: (0, 0)),
        ],
        out_specs=pl.BlockSpec((1, NU, BT), lambda i: (i, 0, 0)),
        out_shape=jax.ShapeDtypeStruct((nb, NU, BT), jnp.int32),
        scratch_shapes=(
            [pltpu.VMEM((BT, K), jnp.float32) for _ in range(NU)]
            + [pltpu.VMEM((BT, 1), jnp.float32) for _ in range(NU)]
        ),
    )(zflat, codebook, cn)
    return out.reshape(t)


DPAD = 128


@functools.lru_cache(maxsize=None)
def _make_sc_gather(t):
    info = plsc.get_sparse_core_info()
    nc, ns = info.num_cores, info.num_subcores
    nw = nc * ns
    bpw = t // nw

    @functools.partial(
        pl.kernel,
        mesh=plsc.VectorSubcoreMesh(core_axis_name="c", subcore_axis_name="s"),
        out_type=jax.ShapeDtypeStruct((t, D), jnp.float32),
        scratch_types=[
            pltpu.VMEM((2, bpw // 2), jnp.int32),
            pltpu.VMEM((2, bpw // 2, D), jnp.float32),
            pltpu.SemaphoreType.DMA,
            pltpu.SemaphoreType.DMA,
        ],
        compiler_params=pltpu.CompilerParams(use_tc_tiling_on_sc=False),
    )
    def gather(cb_hbm, idx_hbm, out_hbm, idx_v, rows_v, sem0, sem1):
        wid = lax.axis_index("s") * nc + lax.axis_index("c")
        half = bpw // 2
        base = wid * bpw
        sems = (sem0, sem1)
        for j in range(2):
            pltpu.sync_copy(idx_hbm.at[pl.ds(base + j * half, half)],
                            idx_v.at[j])
        copies = [
            pltpu.async_copy(cb_hbm.at[idx_v.at[j]], rows_v.at[j], sems[j])
            for j in range(2)
        ]
        for j in range(2):
            copies[j].wait()
            pltpu.sync_copy(rows_v.at[j],
                            out_hbm.at[pl.ds(base + j * half, half)])

    return gather


def kernel(ze, codebook):
    b, s, d = ze.shape
    t = b * s
    zflat = ze.reshape(t, d)
    idx = _tc_argmin(zflat, codebook)
    zq = _make_sc_gather(t)(codebook, idx)
    return zq.reshape(b, s, d)
